# Initial kernel scaffold; baseline (speedup 1.0000x reference)
import jax
import jax.numpy as jnp
from jax.experimental import pallas as pl


def kernel(x, edge_index, W1, b1, W2, b2):
    raise NotImplementedError("write your pallas kernel here")


# speedup vs baseline: 8.2059x; 8.2059x over previous
import functools

import jax
import jax.numpy as jnp
from jax import lax
from jax.experimental import pallas as pl
from jax.experimental.pallas import tpu as pltpu
from jax.experimental.pallas import tpu_sc as plsc

N = 10000
E = 320000
D = 128
NC = 2
NS = 16
L = 16
NW = NC * NS
CHUNK = 128
FULL_ROUNDS = E // (NW * CHUNK)
TAIL_TILES = (E - FULL_ROUNDS * NW * CHUNK) // CHUNK
ROWS_PER_TILE = 624
ROWS_LAST_TILE = N - (NS - 1) * ROWS_PER_TILE
DW = 128
DEG_ROUNDS = E // (NS * CHUNK)
DEG_TAIL_TILES = (E - DEG_ROUNDS * NS * CHUNK) // CHUNK

@functools.cache
def _mesh():
    return plsc.VectorSubcoreMesh(
        core_axis_name="c", subcore_axis_name="s",
        num_cores=NC, num_subcores=NS)


def _zero_rows(ref, nrows, width):
    zero = jnp.zeros((L,), jnp.float32)

    def body(i, _):
        for j in range(width // L):
            ref[i, pl.ds(j * L, L)] = zero
        return 0

    lax.fori_loop(0, nrows, body, 0)


def _zero_acc_slice(acc, src_zero, row0, nrows):
    full, rem = nrows // 128, nrows % 128
    for j in range(full):
        pltpu.sync_copy(src_zero.at[pl.ds(0, 128)],
                        acc.at[pl.ds(row0 + j * 128, 128)])
    if rem:
        pltpu.sync_copy(src_zero.at[pl.ds(0, rem)],
                        acc.at[pl.ds(row0 + full * 128, rem)])


def _per_tile_rows(s, fn):
    @pl.when(s < NS - 1)
    def _():
        fn(ROWS_PER_TILE)

    @pl.when(s == NS - 1)
    def _():
        fn(ROWS_LAST_TILE)


def _degree_body(src_hbm, dst_hbm, out_hbm, ones_v, idx_v, acc):
    c = lax.axis_index("c")
    s = lax.axis_index("s")
    row0 = s * ROWS_PER_TILE

    _zero_rows(ones_v, CHUNK, D)
    _per_tile_rows(s, lambda nr: _zero_acc_slice(acc, ones_v, row0, nr))

    one = jnp.ones((L,), jnp.float32)

    def fill(i, _):
        for j in range(D // L):
            ones_v[i, pl.ds(j * L, L)] = one
        return 0

    lax.fori_loop(0, CHUNK, fill, 0)
    plsc.subcore_barrier()

    def do_chunk(base):
        @pl.when(c == 0)
        def _():
            pltpu.sync_copy(src_hbm.at[pl.ds(base, CHUNK)], idx_v)

        @pl.when(c == 1)
        def _():
            pltpu.sync_copy(dst_hbm.at[pl.ds(base, CHUNK)], idx_v)

        pltpu.sync_copy(ones_v, acc.at[idx_v], add=True)

    def round_body(r, _):
        do_chunk(r * NS * CHUNK + s * CHUNK)
        return 0

    lax.fori_loop(0, DEG_ROUNDS, round_body, 0)

    @pl.when(s < DEG_TAIL_TILES)
    def _():
        do_chunk(DEG_ROUNDS * NS * CHUNK + s * CHUNK)

    plsc.subcore_barrier()
    _per_tile_rows(
        s, lambda nr: pltpu.sync_copy(
            acc.at[pl.ds(row0, nr)],
            out_hbm.at[c, pl.ds(row0, nr)]))


def _aggregate_body(y_hbm, src_hbm, dst_hbm, out_hbm,
                    idx_s, idx_d, rows_v, acc, sem):
    c = lax.axis_index("c")
    s = lax.axis_index("s")
    wid = c * NS + s
    row0 = s * ROWS_PER_TILE

    _zero_rows(rows_v, CHUNK, D)
    _per_tile_rows(s, lambda nr: _zero_acc_slice(acc, rows_v, row0, nr))
    plsc.subcore_barrier()

    def do_chunk(base):
        pltpu.sync_copy(src_hbm.at[pl.ds(base, CHUNK)], idx_s)
        pltpu.sync_copy(dst_hbm.at[pl.ds(base, CHUNK)], idx_d)
        pltpu.async_copy(y_hbm.at[idx_s], rows_v, sem).wait()
        pltpu.sync_copy(rows_v, acc.at[idx_d], add=True)

    def round_body(r, _):
        do_chunk(r * NW * CHUNK + wid * CHUNK)
        return 0

    lax.fori_loop(0, FULL_ROUNDS, round_body, 0)

    @pl.when(wid < TAIL_TILES)
    def _():
        do_chunk(FULL_ROUNDS * NW * CHUNK + wid * CHUNK)

    plsc.subcore_barrier()
    _per_tile_rows(
        s, lambda nr: pltpu.sync_copy(acc.at[pl.ds(row0, nr)],
                                      out_hbm.at[c, pl.ds(row0, nr)]))


@functools.cache
def _degree_kernel():
    return pl.kernel(
        _degree_body,
        out_type=jax.ShapeDtypeStruct((NC, N, DW), jnp.float32),
        mesh=_mesh(),
        scratch_types=[
            pltpu.VMEM((CHUNK, D), jnp.float32),
            pltpu.VMEM((CHUNK,), jnp.int32),
            pltpu.VMEM_SHARED((N, D), jnp.float32),
        ],
    )


@functools.cache
def _aggregate_kernel():
    return pl.kernel(
        _aggregate_body,
        out_type=jax.ShapeDtypeStruct((NC, N, D), jnp.float32),
        mesh=_mesh(),
        scratch_types=[
            pltpu.VMEM((CHUNK,), jnp.int32),
            pltpu.VMEM((CHUNK,), jnp.int32),
            pltpu.VMEM((CHUNK, D), jnp.float32),
            pltpu.VMEM_SHARED((N, D), jnp.float32),
            pltpu.SemaphoreType.DMA,
        ],
    )


BR = 400
GRID = N // BR


def _norms_from(deg_block, kind):
    d = deg_block[kind, :, 0:1]
    return jnp.where(d > 0, lax.rsqrt(d), 0.0)


def _tc1_body(x_ref, w_ref, deg_ref, o_ref):
    deg = deg_ref[...]
    n_src = _norms_from(deg, 0)
    hw = jnp.dot(x_ref[...], w_ref[...], preferred_element_type=jnp.float32)
    o_ref[...] = hw * n_src


def _tc2_body(agg_ref, deg_ref, b_ref, w_ref, o_ref):
    deg = deg_ref[...]
    n_src = _norms_from(deg, 0)
    n_dst = _norms_from(deg, 1)
    h = (agg_ref[0] + agg_ref[1]) * n_dst + b_ref[...]
    hw = jnp.dot(h, w_ref[...], preferred_element_type=jnp.float32)
    o_ref[...] = hw * n_src


def _tc3_body(agg_ref, deg_ref, b_ref, o_ref):
    deg = deg_ref[...]
    n_dst = _norms_from(deg, 1)
    o_ref[...] = (agg_ref[0] + agg_ref[1]) * n_dst + b_ref[...]


_DEG_SPEC = pl.BlockSpec((2, BR, DW), lambda i: (0, i, 0))
_ROW_SPEC = pl.BlockSpec((BR, D), lambda i: (i, 0))
_AGG_SPEC = pl.BlockSpec((NC, BR, D), lambda i: (0, i, 0))
_W_SPEC = pl.BlockSpec((D, D), lambda i: (0, 0))
_B_SPEC = pl.BlockSpec((1, D), lambda i: (0, 0))
_OUT_TYPE = jax.ShapeDtypeStruct((N, D), jnp.float32)


def _tc1(x, W, deg):
    return pl.pallas_call(
        _tc1_body, grid=(GRID,),
        in_specs=[_ROW_SPEC, _W_SPEC, _DEG_SPEC],
        out_specs=_ROW_SPEC, out_shape=_OUT_TYPE,
    )(x, W, deg)


def _tc2(agg, deg, b, W):
    return pl.pallas_call(
        _tc2_body, grid=(GRID,),
        in_specs=[_AGG_SPEC, _DEG_SPEC, _B_SPEC, _W_SPEC],
        out_specs=_ROW_SPEC, out_shape=_OUT_TYPE,
    )(agg, deg, b, W)


def _tc3(agg, deg, b):
    return pl.pallas_call(
        _tc3_body, grid=(GRID,),
        in_specs=[_AGG_SPEC, _DEG_SPEC, _B_SPEC],
        out_specs=_ROW_SPEC, out_shape=_OUT_TYPE,
    )(agg, deg, b)


def kernel(x, edge_index, W1, b1, W2, b2):
    src = edge_index[0].astype(jnp.int32)
    dst = edge_index[1].astype(jnp.int32)
    b1r = b1.reshape(1, D)
    b2r = b2.reshape(1, D)

    deg = _degree_kernel()(src, dst)
    y1 = _tc1(x, W1, deg)
    p1 = _aggregate_kernel()(y1, src, dst)
    y2 = _tc2(p1, deg, b1r, W2)
    p2 = _aggregate_kernel()(y2, src, dst)
    return _tc3(p2, deg, b2r)


# speedup vs baseline: 13.3472x; 1.6265x over previous
import functools

import jax
import jax.numpy as jnp
from jax import lax
from jax.experimental import pallas as pl
from jax.experimental.pallas import tpu as pltpu
from jax.experimental.pallas import tpu_sc as plsc

N = 10000
E = 320000
D = 128
NC = 2
NS = 16
L = 16
NW = NC * NS
CHUNK = 128
CROWS = E // CHUNK
ROWS_PER_TILE = 624
ROWS_LAST_TILE = N - (NS - 1) * ROWS_PER_TILE
DW = 128
DEG_MAIN = CROWS // NS
DEG_EXTRA = CROWS - DEG_MAIN * NS
AGG_MAIN = CROWS // NW
AGG_EXTRA = CROWS - AGG_MAIN * NW
IBLK = 26
DEG_NBLK = DEG_MAIN // IBLK
AGG_NBLK = AGG_MAIN // IBLK

@functools.cache
def _mesh():
    return plsc.VectorSubcoreMesh(
        core_axis_name="c", subcore_axis_name="s",
        num_cores=NC, num_subcores=NS)


def _zero_rows(ref, nrows, width):
    zero = jnp.zeros((L,), jnp.float32)

    def body(i, _):
        for j in range(width // L):
            ref[i, pl.ds(j * L, L)] = zero
        return 0

    lax.fori_loop(0, nrows, body, 0)


def _zero_acc_slice(acc, src_zero, row0, nrows):
    full, rem = nrows // 128, nrows % 128
    for j in range(full):
        pltpu.sync_copy(src_zero.at[pl.ds(0, 128)],
                        acc.at[pl.ds(row0 + j * 128, 128)])
    if rem:
        pltpu.sync_copy(src_zero.at[pl.ds(0, rem)],
                        acc.at[pl.ds(row0 + full * 128, rem)])


def _per_tile_rows(s, fn):
    @pl.when(s < NS - 1)
    def _():
        fn(ROWS_PER_TILE)

    @pl.when(s == NS - 1)
    def _():
        fn(ROWS_LAST_TILE)


def _degree_body(src_hbm, dst_hbm, out_hbm, ones_v, idx_v, acc, sem):
    c = lax.axis_index("c")
    s = lax.axis_index("s")
    row0 = s * ROWS_PER_TILE

    _zero_rows(ones_v, CHUNK, D)
    _per_tile_rows(s, lambda nr: _zero_acc_slice(acc, ones_v, row0, nr))

    one = jnp.ones((L,), jnp.float32)

    def fill(i, _):
        for j in range(D // L):
            ones_v[i, pl.ds(j * L, L)] = one
        return 0

    lax.fori_loop(0, CHUNK, fill, 0)

    plsc.subcore_barrier()

    def drain_one():
        pltpu.make_async_copy(out_hbm.at[0, pl.ds(0, CHUNK)], ones_v,
                              sem).wait()

    def load_idx(base, nrows):
        @pl.when(c == 0)
        def _():
            pltpu.sync_copy(src_hbm.at[pl.ds(base, nrows)],
                            idx_v.at[pl.ds(0, nrows)])

        @pl.when(c == 1)
        def _():
            pltpu.sync_copy(dst_hbm.at[pl.ds(base, nrows)],
                            idx_v.at[pl.ds(0, nrows)])

    def blk_body(t, _):
        load_idx(s * DEG_MAIN + t * IBLK, IBLK)

        def body(j, _):
            pltpu.async_copy(ones_v, acc.at[idx_v.at[j, 0]], sem, add=True)

            @pl.when(j >= 4)
            def _():
                drain_one()

            return 0

        lax.fori_loop(0, IBLK, body, 0)
        for _ in range(4):
            drain_one()
        return 0

    lax.fori_loop(0, DEG_NBLK, blk_body, 0)

    @pl.when(s < DEG_EXTRA)
    def _():
        load_idx(NS * DEG_MAIN + s, 1)
        pltpu.async_copy(ones_v, acc.at[idx_v.at[0, 0]], sem, add=True)
        drain_one()

    plsc.subcore_barrier()
    _per_tile_rows(
        s, lambda nr: pltpu.sync_copy(
            acc.at[pl.ds(row0, nr)],
            out_hbm.at[c, pl.ds(row0, nr)]))


def _aggregate_body(y_hbm, src_hbm, dst_hbm, out_hbm,
                    idx_s, idx_d, rows0, rows1, sg0, sg1, ss0, ss1, acc):
    c = lax.axis_index("c")
    s = lax.axis_index("s")
    wid = c * NS + s
    row0 = s * ROWS_PER_TILE

    _zero_rows(rows0, CHUNK, D)
    _per_tile_rows(s, lambda nr: _zero_acc_slice(acc, rows0, row0, nr))

    plsc.subcore_barrier()

    bufs = (rows0, rows1)
    gsems = (sg0, sg1)
    ssems = (ss0, ss1)

    def g_start(j, b):
        pltpu.async_copy(y_hbm.at[idx_s.at[j, 0]], bufs[b], gsems[b])

    def g_wait(b):
        pltpu.make_async_copy(y_hbm.at[pl.ds(0, CHUNK)], bufs[b],
                              gsems[b]).wait()

    def s_start(j, b):
        pltpu.async_copy(bufs[b], acc.at[idx_d.at[j, 0]], ssems[b], add=True)

    def s_wait(b):
        pltpu.make_async_copy(y_hbm.at[pl.ds(0, CHUNK)], bufs[b],
                              ssems[b]).wait()

    def blk_body(t, _):
        base = wid * AGG_MAIN + t * IBLK
        pltpu.sync_copy(src_hbm.at[pl.ds(base, IBLK)],
                        idx_s.at[pl.ds(0, IBLK)])
        pltpu.sync_copy(dst_hbm.at[pl.ds(base, IBLK)],
                        idx_d.at[pl.ds(0, IBLK)])
        g_start(0, 0)

        def body(j, _):
            def per_parity(b):
                g_wait(b)

                @pl.when(j + 1 < IBLK)
                def _():
                    @pl.when(j >= 1)
                    def _():
                        s_wait(1 - b)

                    g_start(j + 1, 1 - b)

                s_start(j, b)

            @pl.when(j % 2 == 0)
            def _():
                per_parity(0)

            @pl.when(j % 2 == 1)
            def _():
                per_parity(1)

            return 0

        lax.fori_loop(0, IBLK, body, 0)
        s_wait(0)
        s_wait(1)
        return 0

    lax.fori_loop(0, AGG_NBLK, blk_body, 0)

    @pl.when(wid < AGG_EXTRA)
    def _():
        pltpu.sync_copy(src_hbm.at[pl.ds(NW * AGG_MAIN + wid, 1)],
                        idx_s.at[pl.ds(0, 1)])
        pltpu.sync_copy(dst_hbm.at[pl.ds(NW * AGG_MAIN + wid, 1)],
                        idx_d.at[pl.ds(0, 1)])
        g_start(0, 0)
        g_wait(0)
        s_start(0, 0)
        s_wait(0)

    plsc.subcore_barrier()
    _per_tile_rows(
        s, lambda nr: pltpu.sync_copy(acc.at[pl.ds(row0, nr)],
                                      out_hbm.at[c, pl.ds(row0, nr)]))


@functools.cache
def _degree_kernel():
    return pl.kernel(
        _degree_body,
        out_type=jax.ShapeDtypeStruct((NC, N, DW), jnp.float32),
        mesh=_mesh(),
        scratch_types=[
            pltpu.VMEM((CHUNK, D), jnp.float32),
            pltpu.VMEM((IBLK, 1, CHUNK), jnp.int32),
            pltpu.VMEM_SHARED((N, D), jnp.float32),
            pltpu.SemaphoreType.DMA,
        ],
    )


@functools.cache
def _aggregate_kernel():
    return pl.kernel(
        _aggregate_body,
        out_type=jax.ShapeDtypeStruct((NC, N, D), jnp.float32),
        mesh=_mesh(),
        scratch_types=[
            pltpu.VMEM((IBLK, 1, CHUNK), jnp.int32),
            pltpu.VMEM((IBLK, 1, CHUNK), jnp.int32),
            pltpu.VMEM((CHUNK, D), jnp.float32),
            pltpu.VMEM((CHUNK, D), jnp.float32),
            pltpu.SemaphoreType.DMA,
            pltpu.SemaphoreType.DMA,
            pltpu.SemaphoreType.DMA,
            pltpu.SemaphoreType.DMA,
            pltpu.VMEM_SHARED((N, D), jnp.float32),
        ],
    )


BR = 400
GRID = N // BR


def _norms_from(deg_block, kind):
    d = deg_block[kind, :, 0:1]
    return jnp.where(d > 0, lax.rsqrt(d), 0.0)


def _tc1_body(x_ref, w_ref, deg_ref, o_ref):
    deg = deg_ref[...]
    n_src = _norms_from(deg, 0)
    hw = jnp.dot(x_ref[...], w_ref[...], preferred_element_type=jnp.float32)
    o_ref[...] = hw * n_src


def _tc2_body(agg_ref, deg_ref, b_ref, w_ref, o_ref):
    deg = deg_ref[...]
    n_src = _norms_from(deg, 0)
    n_dst = _norms_from(deg, 1)
    h = (agg_ref[0] + agg_ref[1]) * n_dst + b_ref[...]
    hw = jnp.dot(h, w_ref[...], preferred_element_type=jnp.float32)
    o_ref[...] = hw * n_src


def _tc3_body(agg_ref, deg_ref, b_ref, o_ref):
    deg = deg_ref[...]
    n_dst = _norms_from(deg, 1)
    o_ref[...] = (agg_ref[0] + agg_ref[1]) * n_dst + b_ref[...]


_DEG_SPEC = pl.BlockSpec((2, BR, DW), lambda i: (0, i, 0))
_ROW_SPEC = pl.BlockSpec((BR, D), lambda i: (i, 0))
_AGG_SPEC = pl.BlockSpec((NC, BR, D), lambda i: (0, i, 0))
_W_SPEC = pl.BlockSpec((D, D), lambda i: (0, 0))
_B_SPEC = pl.BlockSpec((1, D), lambda i: (0, 0))
_OUT_TYPE = jax.ShapeDtypeStruct((N, D), jnp.float32)


def _tc1(x, W, deg):
    return pl.pallas_call(
        _tc1_body, grid=(GRID,),
        in_specs=[_ROW_SPEC, _W_SPEC, _DEG_SPEC],
        out_specs=_ROW_SPEC, out_shape=_OUT_TYPE,
    )(x, W, deg)


def _tc2(agg, deg, b, W):
    return pl.pallas_call(
        _tc2_body, grid=(GRID,),
        in_specs=[_AGG_SPEC, _DEG_SPEC, _B_SPEC, _W_SPEC],
        out_specs=_ROW_SPEC, out_shape=_OUT_TYPE,
    )(agg, deg, b, W)


def _tc3(agg, deg, b):
    return pl.pallas_call(
        _tc3_body, grid=(GRID,),
        in_specs=[_AGG_SPEC, _DEG_SPEC, _B_SPEC],
        out_specs=_ROW_SPEC, out_shape=_OUT_TYPE,
    )(agg, deg, b)


def kernel(x, edge_index, W1, b1, W2, b2):
    src = edge_index[0].astype(jnp.int32).reshape(CROWS, 1, CHUNK)
    dst = edge_index[1].astype(jnp.int32).reshape(CROWS, 1, CHUNK)
    b1r = b1.reshape(1, D)
    b2r = b2.reshape(1, D)

    deg = _degree_kernel()(src, dst)
    y1 = _tc1(x, W1, deg)
    p1 = _aggregate_kernel()(y1, src, dst)
    y2 = _tc2(p1, deg, b1r, W2)
    p2 = _aggregate_kernel()(y2, src, dst)
    return _tc3(p2, deg, b2r)
